# Initial kernel scaffold; baseline (speedup 1.0000x reference)
import jax
import jax.numpy as jnp
from jax.experimental import pallas as pl


def kernel(x, emb1, emb2):
    raise NotImplementedError("write your pallas kernel here")


# speedup vs baseline: 1.5734x; 1.5734x over previous
import jax
import jax.numpy as jnp
from jax.experimental import pallas as pl
from jax.experimental.pallas import tpu as pltpu

_K = 1024
_D = 256
_SUB = 128
_B = 16
_T = 1024
_N = _B * _T
_BETA = 0.25


def _vq_body(x_ref, e1_ref, e2_ref, cs1_ref, cs2_ref, is1_ref, is2_ref,
             out_ref, loss_ref, perp_ref, cnt1_ref, cnt2_ref, sq_ref):
    b = pl.program_id(0)
    xb = x_ref[0]
    x1 = xb[:_SUB, :]
    x2 = xb[_SUB:, :]
    e1 = e1_ref[...]
    e2 = e2_ref[...]
    is1 = is1_ref[0]
    is2 = is2_ref[0]

    def half(e, cs_ref, xh, is_row):
        mm = jax.lax.dot_general(e, xh, (((1,), (0,)), ((), ())),
                                 preferred_element_type=jnp.float32)
        dis = (cs_ref[...] + is_row) - 2.0 * mm
        md = jnp.min(dis, axis=0, keepdims=True)
        iota = jax.lax.broadcasted_iota(jnp.int32, (_K, _T), 0)
        ind = jnp.min(jnp.where(dis == md, iota, _K),
                      axis=0, keepdims=True)
        oh = jnp.where(iota == ind, 1.0, 0.0)
        q = jax.lax.dot_general(e, oh, (((0,), (0,)), ((), ())),
                                preferred_element_type=jnp.float32)
        cnt = jnp.sum(oh, axis=1, keepdims=True)
        return q, cnt

    q1, c1 = half(e1, cs1_ref, x1, is1)
    q2, c2 = half(e2, cs2_ref, x2, is2)

    out_ref[0, :_SUB, :] = q1
    out_ref[0, _SUB:, :] = q2

    s = jnp.sum((q1 - x1) ** 2, keepdims=True) + jnp.sum((q2 - x2) ** 2, keepdims=True)

    @pl.when(b == 0)
    def _():
        cnt1_ref[...] = c1
        cnt2_ref[...] = c2
        sq_ref[...] = s

    @pl.when(b > 0)
    def _():
        cnt1_ref[...] += c1
        cnt2_ref[...] += c2
        sq_ref[...] += s

    @pl.when(b == _B - 1)
    def _():
        mse = sq_ref[...] * (1.0 / float(_N * _D))
        loss_ref[...] = mse + _BETA * mse
        p1 = cnt1_ref[...] * (1.0 / float(_N))
        p2 = cnt2_ref[...] * (1.0 / float(_N))
        s1 = jnp.sum(p1 * jnp.log(p1 + 1e-10), keepdims=True)
        s2 = jnp.sum(p2 * jnp.log(p2 + 1e-10), keepdims=True)
        perp_ref[...] = jnp.exp(-1.0 * s1) + jnp.exp(-1.0 * s2)


def kernel(x, emb1, emb2):
    xp = jnp.transpose(x, (0, 2, 1))
    flat_in = xp.reshape(-1, _D)
    in_sqr1 = jnp.sum(flat_in[:, :_SUB] ** 2, axis=1, keepdims=True)
    in_sqr2 = jnp.sum(flat_in[:, _SUB:] ** 2, axis=1, keepdims=True)
    cs1 = jnp.sum(emb1 ** 2, axis=1).reshape(_K, 1)
    cs2 = jnp.sum(emb2 ** 2, axis=1).reshape(_K, 1)
    is1 = in_sqr1.reshape(_B, 1, _T)
    is2 = in_sqr2.reshape(_B, 1, _T)

    out, loss, perp = pl.pallas_call(
        _vq_body,
        grid=(_B,),
        in_specs=[
            pl.BlockSpec((1, _D, _T), lambda b: (b, 0, 0)),
            pl.BlockSpec((_K, _SUB), lambda b: (0, 0)),
            pl.BlockSpec((_K, _SUB), lambda b: (0, 0)),
            pl.BlockSpec((_K, 1), lambda b: (0, 0)),
            pl.BlockSpec((_K, 1), lambda b: (0, 0)),
            pl.BlockSpec((1, 1, _T), lambda b: (b, 0, 0)),
            pl.BlockSpec((1, 1, _T), lambda b: (b, 0, 0)),
        ],
        out_specs=[
            pl.BlockSpec((1, _D, _T), lambda b: (b, 0, 0)),
            pl.BlockSpec((1, 1), lambda b: (0, 0)),
            pl.BlockSpec((1, 1), lambda b: (0, 0)),
        ],
        out_shape=[
            jax.ShapeDtypeStruct((_B, _D, _T), jnp.float32),
            jax.ShapeDtypeStruct((1, 1), jnp.float32),
            jax.ShapeDtypeStruct((1, 1), jnp.float32),
        ],
        scratch_shapes=[
            pltpu.VMEM((_K, 1), jnp.float32),
            pltpu.VMEM((_K, 1), jnp.float32),
            pltpu.VMEM((1, 1), jnp.float32),
        ],
        interpret=False,
    )(x, emb1, emb2, cs1, cs2, is1, is2)
    return out, loss[0, 0], perp[0, 0]


# speedup vs baseline: 2.0616x; 1.3102x over previous
import jax
import jax.numpy as jnp
from jax.experimental import pallas as pl
from jax.experimental.pallas import tpu as pltpu

_K = 1024
_D = 256
_SUB = 128
_B = 16
_T = 1024
_N = _B * _T
_BETA = 0.25


def _vq_body(x_ref, e1_ref, e2_ref, cs1_ref, cs2_ref, is1_ref, is2_ref,
             out_ref, loss_ref, perp_ref, cnt1_ref, cnt2_ref, sq_ref):
    b = pl.program_id(0)
    xb = x_ref[0]
    x1 = xb[:_SUB, :]
    x2 = xb[_SUB:, :]
    e1 = e1_ref[...]
    e2 = e2_ref[...]
    is1 = is1_ref[0]
    is2 = is2_ref[0]

    def half(e, cs_ref, xh, is_row):
        mm = jax.lax.dot_general(e, xh, (((1,), (0,)), ((), ())),
                                 preferred_element_type=jnp.float32)
        dis = (cs_ref[...] + is_row) - 2.0 * mm
        md = jnp.min(dis, axis=0, keepdims=True)
        iota = jax.lax.broadcasted_iota(jnp.int32, (_K, _T), 0)
        ind = jnp.min(jnp.where(dis == md, iota, _K),
                      axis=0, keepdims=True)
        oh = jnp.where(iota == ind, 1.0, 0.0)
        q = jax.lax.dot_general(e, oh, (((0,), (0,)), ((), ())),
                                preferred_element_type=jnp.float32)
        cnt = jnp.sum(oh, axis=1, keepdims=True)
        return q, cnt

    q1, c1 = half(e1, cs1_ref, x1, is1)
    q2, c2 = half(e2, cs2_ref, x2, is2)

    out_ref[0, :_SUB, :] = q1
    out_ref[0, _SUB:, :] = q2

    s = jnp.sum((q1 - x1) ** 2, keepdims=True) + jnp.sum((q2 - x2) ** 2, keepdims=True)

    @pl.when(b == 0)
    def _():
        cnt1_ref[...] = c1
        cnt2_ref[...] = c2
        sq_ref[...] = s

    @pl.when(b > 0)
    def _():
        cnt1_ref[...] += c1
        cnt2_ref[...] += c2
        sq_ref[...] += s

    @pl.when(b == _B - 1)
    def _():
        mse = sq_ref[...] * (1.0 / float(_N * _D))
        loss_ref[...] = mse + _BETA * mse
        p1 = cnt1_ref[...] * (1.0 / float(_N))
        p2 = cnt2_ref[...] * (1.0 / float(_N))
        s1 = jnp.sum(p1 * jnp.log(p1 + 1e-10), keepdims=True)
        s2 = jnp.sum(p2 * jnp.log(p2 + 1e-10), keepdims=True)
        perp_ref[...] = jnp.exp(-1.0 * s1) + jnp.exp(-1.0 * s2)


def kernel(x, emb1, emb2):
    xp = jnp.transpose(x, (0, 2, 1))
    in_sqr1 = jnp.sum(xp[:, :, :_SUB] ** 2, axis=2)
    in_sqr2 = jnp.sum(xp[:, :, _SUB:] ** 2, axis=2)
    cs1 = jnp.sum(emb1 ** 2, axis=1).reshape(_K, 1)
    cs2 = jnp.sum(emb2 ** 2, axis=1).reshape(_K, 1)
    is1 = in_sqr1.reshape(_B, 1, _T)
    is2 = in_sqr2.reshape(_B, 1, _T)

    out, loss, perp = pl.pallas_call(
        _vq_body,
        grid=(_B,),
        in_specs=[
            pl.BlockSpec((1, _D, _T), lambda b: (b, 0, 0)),
            pl.BlockSpec((_K, _SUB), lambda b: (0, 0)),
            pl.BlockSpec((_K, _SUB), lambda b: (0, 0)),
            pl.BlockSpec((_K, 1), lambda b: (0, 0)),
            pl.BlockSpec((_K, 1), lambda b: (0, 0)),
            pl.BlockSpec((1, 1, _T), lambda b: (b, 0, 0)),
            pl.BlockSpec((1, 1, _T), lambda b: (b, 0, 0)),
        ],
        out_specs=[
            pl.BlockSpec((1, _D, _T), lambda b: (b, 0, 0)),
            pl.BlockSpec((1, 1), lambda b: (0, 0)),
            pl.BlockSpec((1, 1), lambda b: (0, 0)),
        ],
        out_shape=[
            jax.ShapeDtypeStruct((_B, _D, _T), jnp.float32),
            jax.ShapeDtypeStruct((1, 1), jnp.float32),
            jax.ShapeDtypeStruct((1, 1), jnp.float32),
        ],
        scratch_shapes=[
            pltpu.VMEM((_K, 1), jnp.float32),
            pltpu.VMEM((_K, 1), jnp.float32),
            pltpu.VMEM((1, 1), jnp.float32),
        ],
        interpret=False,
    )(x, emb1, emb2, cs1, cs2, is1, is2)
    return out, loss[0, 0], perp[0, 0]


# speedup vs baseline: 2.4320x; 1.1797x over previous
import jax
import jax.numpy as jnp
from jax.experimental import pallas as pl
from jax.experimental.pallas import tpu as pltpu

_K = 1024
_D = 256
_SUB = 128
_B = 16
_T = 1024
_N = _B * _T
_BETA = 0.25


def _vq_body(x_ref, e1_ref, e2_ref, cs1_ref, cs2_ref, is1_ref, is2_ref,
             out_ref, loss_ref, perp_ref, cnt1_ref, cnt2_ref, sq_ref):
    b = pl.program_id(0)
    xb = x_ref[0]
    x1 = xb[:_SUB, :]
    x2 = xb[_SUB:, :]
    e1 = e1_ref[...]
    e2 = e2_ref[...]
    is1 = is1_ref[0]
    is2 = is2_ref[0]

    iota_f = jax.lax.broadcasted_iota(jnp.int32, (_K, _T), 0).astype(jnp.float32)

    def half(e, cs_ref, xh, is_row):
        mm = jax.lax.dot_general(e, xh, (((1,), (0,)), ((), ())),
                                 preferred_element_type=jnp.float32)
        dis = (cs_ref[...] + is_row) - 2.0 * mm
        md = jnp.min(dis, axis=0, keepdims=True)
        ind = jnp.argmin(dis, axis=0).astype(jnp.float32)[None, :]
        oh = jnp.where(iota_f == ind, 1.0, 0.0)
        q = jax.lax.dot_general(e, oh, (((0,), (0,)), ((), ())),
                                preferred_element_type=jnp.float32)
        cnt = jnp.sum(oh, axis=1, keepdims=True)
        return q, cnt, md

    q1, c1, md1 = half(e1, cs1_ref, x1, is1)
    q2, c2, md2 = half(e2, cs2_ref, x2, is2)

    out_ref[0, :_SUB, :] = q1
    out_ref[0, _SUB:, :] = q2

    s = jnp.sum(md1, keepdims=True) + jnp.sum(md2, keepdims=True)

    @pl.when(b == 0)
    def _():
        cnt1_ref[...] = c1
        cnt2_ref[...] = c2
        sq_ref[...] = s

    @pl.when(b > 0)
    def _():
        cnt1_ref[...] += c1
        cnt2_ref[...] += c2
        sq_ref[...] += s

    @pl.when(b == _B - 1)
    def _():
        mse = sq_ref[...] * (1.0 / float(_N * _D))
        loss_ref[...] = mse + _BETA * mse
        p1 = cnt1_ref[...] * (1.0 / float(_N))
        p2 = cnt2_ref[...] * (1.0 / float(_N))
        s1 = jnp.sum(p1 * jnp.log(p1 + 1e-10), keepdims=True)
        s2 = jnp.sum(p2 * jnp.log(p2 + 1e-10), keepdims=True)
        perp_ref[...] = jnp.exp(-1.0 * s1) + jnp.exp(-1.0 * s2)


def kernel(x, emb1, emb2):
    xp = jnp.transpose(x, (0, 2, 1))
    in_sqr1 = jnp.sum(xp[:, :, :_SUB] ** 2, axis=2)
    in_sqr2 = jnp.sum(xp[:, :, _SUB:] ** 2, axis=2)
    cs1 = jnp.sum(emb1 ** 2, axis=1).reshape(_K, 1)
    cs2 = jnp.sum(emb2 ** 2, axis=1).reshape(_K, 1)
    is1 = in_sqr1.reshape(_B, 1, _T)
    is2 = in_sqr2.reshape(_B, 1, _T)

    out, loss, perp = pl.pallas_call(
        _vq_body,
        grid=(_B,),
        in_specs=[
            pl.BlockSpec((1, _D, _T), lambda b: (b, 0, 0)),
            pl.BlockSpec((_K, _SUB), lambda b: (0, 0)),
            pl.BlockSpec((_K, _SUB), lambda b: (0, 0)),
            pl.BlockSpec((_K, 1), lambda b: (0, 0)),
            pl.BlockSpec((_K, 1), lambda b: (0, 0)),
            pl.BlockSpec((1, 1, _T), lambda b: (b, 0, 0)),
            pl.BlockSpec((1, 1, _T), lambda b: (b, 0, 0)),
        ],
        out_specs=[
            pl.BlockSpec((1, _D, _T), lambda b: (b, 0, 0)),
            pl.BlockSpec((1, 1), lambda b: (0, 0)),
            pl.BlockSpec((1, 1), lambda b: (0, 0)),
        ],
        out_shape=[
            jax.ShapeDtypeStruct((_B, _D, _T), jnp.float32),
            jax.ShapeDtypeStruct((1, 1), jnp.float32),
            jax.ShapeDtypeStruct((1, 1), jnp.float32),
        ],
        scratch_shapes=[
            pltpu.VMEM((_K, 1), jnp.float32),
            pltpu.VMEM((_K, 1), jnp.float32),
            pltpu.VMEM((1, 1), jnp.float32),
        ],
        interpret=False,
    )(x, emb1, emb2, cs1, cs2, is1, is2)
    return out, loss[0, 0], perp[0, 0]
